# baseline (speedup 1.0000x reference)
import functools

import jax
import jax.numpy as jnp
from jax import lax
from jax.experimental import pallas as pl
from jax.experimental.pallas import tpu as pltpu
from jax.experimental.pallas import tpu_sc as plsc

N = 10000
H = 32
C = 128
K = 43
KP = 48
E = N * H

S = 2
ES = E // S
NS = N // S

NW = 32
CHUNK = 40
NCHUNK = ES // (NW * CHUNK)
EPW = ES // NW

NB = 200
GB = NS // NB


def _fc1_bn_relu_body(feat_ref, w_ref, g_ref, b_ref, o_ref):
    y = jnp.dot(feat_ref[...], w_ref[...], preferred_element_type=jnp.float32)
    m = jnp.mean(y, axis=0, keepdims=True)
    v = jnp.mean((y - m) * (y - m), axis=0, keepdims=True)
    xn = (y - m) * lax.rsqrt(v + 1e-5) * g_ref[...] + b_ref[...]
    o_ref[...] = jnp.maximum(xn, 0.0)


def _epilogue_body(conv_ref, feat_ref, w_ref, g2_ref, b2_ref, g3_ref, b3_ref,
                   o_ref):
    y = conv_ref[...]
    m = jnp.mean(y, axis=0, keepdims=True)
    v = jnp.mean((y - m) * (y - m), axis=0, keepdims=True)
    y = jnp.maximum((y - m) * lax.rsqrt(v + 1e-5) * g2_ref[...] + b2_ref[...],
                    0.0)
    z = jnp.dot(y, w_ref[...], preferred_element_type=jnp.float32)
    m = jnp.mean(z, axis=0, keepdims=True)
    v = jnp.mean((z - m) * (z - m), axis=0, keepdims=True)
    z = (z - m) * lax.rsqrt(v + 1e-5) * g3_ref[...] + b3_ref[...]
    o_ref[...] = jnp.maximum(feat_ref[...] + z, 0.0)


def _kpconv_body(gx_ref, rel_ref, m5_ref, kpw_ref, o_ref):
    rel = rel_ref[...]
    sq = jnp.sum(rel * rel, axis=0, keepdims=True)
    r5 = jnp.concatenate(
        [rel, sq, jnp.ones((1, NB * H), jnp.float32)], axis=0)
    d2 = jax.lax.dot_general(
        r5, m5_ref[0:5, :], (((0,), (0,)), ((), ())),
        preferred_element_type=jnp.float32,
        precision=jax.lax.Precision.HIGHEST)
    d2 = jnp.maximum(d2, 0.0) + 1e-12
    infl = jnp.maximum(1.0 - d2 * jax.lax.rsqrt(d2), 0.0)
    a = jnp.dot(infl, kpw_ref[...], preferred_element_type=jnp.float32)
    contrib = a * gx_ref[...]
    o_ref[...] = jnp.sum(contrib.reshape(NB, H, C), axis=1)


def _sc_gather_body(x_hbm, cx_hbm, cy_hbm, cz_hbm, idx_hbm,
                    gx_hbm, rel_hbm,
                    idx_v, xrow_v, cx_v, cy_v, cz_v, rx_v, ry_v, rz_v, sem1):
    c = lax.axis_index("c")
    s = lax.axis_index("s")
    wid = s * 2 + c
    pltpu.sync_copy(idx_hbm.at[wid], idx_v)
    pltpu.sync_copy(cx_hbm, cx_v)
    pltpu.sync_copy(cy_hbm, cy_v)
    pltpu.sync_copy(cz_hbm, cz_v)
    lanes = lax.iota(jnp.int32, 16)

    def body(i, carry):
        base = pl.multiple_of(wid * EPW + i * CHUNK, 8)
        cp1 = pltpu.async_copy(x_hbm.at[idx_v.at[i]], xrow_v, sem1)
        for v in range(CHUNK // 16):
            j = idx_v[i, pl.ds(v * 16, 16)]
            n = jax.lax.shift_right_logical(base + v * 16 + lanes, 5)
            off = i * CHUNK + v * 16
            rx_v[pl.ds(off, 16)] = (plsc.load_gather(cx_v, [j])
                                    - plsc.load_gather(cx_v, [n]))
            ry_v[pl.ds(off, 16)] = (plsc.load_gather(cy_v, [j])
                                    - plsc.load_gather(cy_v, [n]))
            rz_v[pl.ds(off, 16)] = (plsc.load_gather(cz_v, [j])
                                    - plsc.load_gather(cz_v, [n]))
        cp1.wait()
        pltpu.sync_copy(xrow_v, gx_hbm.at[pl.ds(base, CHUNK)])
        return carry

    lax.fori_loop(0, NCHUNK, body, 0)
    base = pl.multiple_of(wid * EPW, 8)
    pltpu.sync_copy(rx_v, rel_hbm.at[pl.ds(base, EPW)])
    pltpu.sync_copy(ry_v, rel_hbm.at[pl.ds(ES + base, EPW)])
    pltpu.sync_copy(rz_v, rel_hbm.at[pl.ds(2 * ES + base, EPW)])


def _make_sc_gather():
    f32 = jnp.float32
    return functools.partial(
        pl.kernel,
        out_type=[
            jax.ShapeDtypeStruct((ES, C), f32),
            jax.ShapeDtypeStruct((3 * ES,), f32),
        ],
        mesh=plsc.VectorSubcoreMesh(core_axis_name="c", subcore_axis_name="s"),
        compiler_params=pltpu.CompilerParams(needs_layout_passes=False),
        scratch_types=[
            pltpu.VMEM((NCHUNK, CHUNK), jnp.int32),
            pltpu.VMEM((CHUNK, C), f32),
            pltpu.VMEM((N,), f32),
            pltpu.VMEM((N,), f32),
            pltpu.VMEM((N,), f32),
            pltpu.VMEM((EPW,), f32),
            pltpu.VMEM((EPW,), f32),
            pltpu.VMEM((EPW,), f32),
            pltpu.SemaphoreType.DMA,
        ],
    )(_sc_gather_body)


def kernel(coord, feat, W1, W3, kernel_points, kp_weights,
           g1, b1, g2, b2, g3, b3, reference_index):
    f32 = jnp.float32
    g1 = g1.reshape(1, C); b1 = b1.reshape(1, C)
    g2 = g2.reshape(1, C); b2 = b2.reshape(1, C)
    g3 = g3.reshape(1, C); b3 = b3.reshape(1, C)

    x = pl.pallas_call(
        _fc1_bn_relu_body,
        out_shape=jax.ShapeDtypeStruct((N, C), f32),
    )(feat, W1.T, g1, b1)

    idx = reference_index.astype(jnp.int32).reshape(S, NW, NCHUNK, CHUNK)
    ct = coord.T
    cx, cy, cz = ct[0], ct[1], ct[2]
    kn2 = jnp.sum(kernel_points * kernel_points, axis=1)
    m5 = jnp.zeros((8, KP), f32)
    m5 = m5.at[0:3, :K].set(-2.0 * kernel_points.T)
    m5 = m5.at[3, :].set(1.0)
    m5 = m5.at[4, :K].set(kn2)
    kpw = jnp.zeros((KP, C), f32).at[:K, :].set(kp_weights)

    sc_gather = _make_sc_gather()
    gathered = [sc_gather(x, cx, cy, cz, idx[s]) for s in range(S)]
    conv_parts = []
    for s in range(S):
        gx, rel1d = gathered[s]
        conv_parts.append(pl.pallas_call(
            _kpconv_body,
            grid=(GB,),
            in_specs=[
                pl.BlockSpec((NB * H, C), lambda i: (i, 0)),
                pl.BlockSpec((3, NB * H), lambda i: (0, i)),
                pl.BlockSpec((8, KP), lambda i: (0, 0)),
                pl.BlockSpec((KP, C), lambda i: (0, 0)),
            ],
            out_specs=pl.BlockSpec((NB, C), lambda i: (i, 0)),
            out_shape=jax.ShapeDtypeStruct((NS, C), f32),
        )(gx, rel1d.reshape(3, ES), m5, kpw))
    conv = jnp.concatenate(conv_parts, axis=0)

    out = pl.pallas_call(
        _epilogue_body,
        out_shape=jax.ShapeDtypeStruct((N, C), f32),
    )(conv, feat, W3.T, g2, b2, g3, b3)
    return out


# speedup vs baseline: 1.5874x; 1.5874x over previous
import functools

import jax
import jax.numpy as jnp
from jax import lax
from jax.experimental import pallas as pl
from jax.experimental.pallas import tpu as pltpu
from jax.experimental.pallas import tpu_sc as plsc

N = 10000
H = 32
C = 128
K = 43
KP = 48
E = N * H

NW = 32
CHUNK = 80
NCHUNK = E // (NW * CHUNK)
EPW = E // NW
NBUF = 5

NB = 400
GB = N // NB


def _fc1_bn_relu_body(feat_ref, w_ref, g_ref, b_ref, o_ref):
    y = jnp.dot(feat_ref[...], w_ref[...], preferred_element_type=jnp.float32)
    m = jnp.mean(y, axis=0, keepdims=True)
    v = jnp.mean((y - m) * (y - m), axis=0, keepdims=True)
    xn = (y - m) * lax.rsqrt(v + 1e-5) * g_ref[...] + b_ref[...]
    o_ref[...] = jnp.maximum(xn, 0.0)


def _epilogue_body(conv_ref, feat_ref, w_ref, g2_ref, b2_ref, g3_ref, b3_ref,
                   o_ref):
    y = conv_ref[...]
    m = jnp.mean(y, axis=0, keepdims=True)
    v = jnp.mean((y - m) * (y - m), axis=0, keepdims=True)
    y = jnp.maximum((y - m) * lax.rsqrt(v + 1e-5) * g2_ref[...] + b2_ref[...],
                    0.0)
    z = jnp.dot(y, w_ref[...], preferred_element_type=jnp.float32)
    m = jnp.mean(z, axis=0, keepdims=True)
    v = jnp.mean((z - m) * (z - m), axis=0, keepdims=True)
    z = (z - m) * lax.rsqrt(v + 1e-5) * g3_ref[...] + b3_ref[...]
    o_ref[...] = jnp.maximum(feat_ref[...] + z, 0.0)


def _kpconv_body(gx_ref, rel_ref, m5_ref, kpw_ref, o_ref):
    rel = rel_ref[...]
    sq = jnp.sum(rel * rel, axis=0, keepdims=True)
    r5 = jnp.concatenate(
        [rel, sq, jnp.ones((1, NB * H), jnp.float32)], axis=0)
    d2 = jax.lax.dot_general(
        r5, m5_ref[0:5, :], (((0,), (0,)), ((), ())),
        preferred_element_type=jnp.float32,
        precision=jax.lax.Precision.HIGHEST)
    d2 = jnp.maximum(d2, 0.0) + 1e-12
    infl = jnp.maximum(1.0 - d2 * jax.lax.rsqrt(d2), 0.0)
    a = jnp.dot(infl, kpw_ref[...], preferred_element_type=jnp.float32)
    contrib = a * gx_ref[...]
    o_ref[...] = jnp.sum(contrib.reshape(NB, H, C), axis=1)


def _sc_gather_body(x_hbm, cx_hbm, cy_hbm, cz_hbm, idx_hbm,
                    gx_hbm, rel_hbm,
                    idx_v, xrow_v, cx_v, cy_v, cz_v, rx_v, ry_v, rz_v, *sems):
    gsems = sems[:NBUF]
    wsems = sems[NBUF:]
    c = lax.axis_index("c")
    s = lax.axis_index("s")
    wid = s * 2 + c
    pltpu.sync_copy(idx_hbm.at[wid], idx_v)
    pltpu.sync_copy(cx_hbm, cx_v)
    pltpu.sync_copy(cy_hbm, cy_v)
    pltpu.sync_copy(cz_hbm, cz_v)
    lanes = lax.iota(jnp.int32, 16)
    wbase = pl.multiple_of(wid * EPW, 8)

    def g_start(b, i):
        pltpu.async_copy(x_hbm.at[idx_v.at[i]], xrow_v.at[b], gsems[b])

    for b in range(NBUF):
        g_start(b, b)

    def body(i5, carry):
        for b in range(NBUF):
            i = i5 * NBUF + b
            for v in range(CHUNK // 16):
                j = idx_v[i, pl.ds(v * 16, 16)]
                n = jax.lax.shift_right_logical(
                    wbase + i * CHUNK + v * 16 + lanes, 5)
                off = i * CHUNK + v * 16
                rx_v[pl.ds(off, 16)] = (plsc.load_gather(cx_v, [j])
                                        - plsc.load_gather(cx_v, [n]))
                ry_v[pl.ds(off, 16)] = (plsc.load_gather(cy_v, [j])
                                        - plsc.load_gather(cy_v, [n]))
                rz_v[pl.ds(off, 16)] = (plsc.load_gather(cz_v, [j])
                                        - plsc.load_gather(cz_v, [n]))
            pltpu.make_async_copy(
                x_hbm.at[pl.ds(0, CHUNK)], xrow_v.at[b], gsems[b]).wait()
            pltpu.async_copy(
                xrow_v.at[b],
                gx_hbm.at[pl.ds(wbase + i * CHUNK, CHUNK)], wsems[b])
        for b in range(NBUF):
            i_next = (i5 + 1) * NBUF + b
            pltpu.make_async_copy(
                xrow_v.at[b], gx_hbm.at[pl.ds(0, CHUNK)], wsems[b]).wait()

            @pl.when(i_next < NCHUNK)
            def _():
                g_start(b, i_next)

        return carry

    lax.fori_loop(0, NCHUNK // NBUF, body, 0)
    pltpu.sync_copy(rx_v, rel_hbm.at[pl.ds(wbase, EPW)])
    pltpu.sync_copy(ry_v, rel_hbm.at[pl.ds(E + wbase, EPW)])
    pltpu.sync_copy(rz_v, rel_hbm.at[pl.ds(2 * E + wbase, EPW)])


def _make_sc_gather():
    f32 = jnp.float32
    return functools.partial(
        pl.kernel,
        out_type=[
            jax.ShapeDtypeStruct((E, C), f32),
            jax.ShapeDtypeStruct((3 * E,), f32),
        ],
        mesh=plsc.VectorSubcoreMesh(core_axis_name="c", subcore_axis_name="s"),
        compiler_params=pltpu.CompilerParams(needs_layout_passes=False),
        scratch_types=(
            [
                pltpu.VMEM((NCHUNK, CHUNK), jnp.int32),
                pltpu.VMEM((NBUF, CHUNK, C), f32),
                pltpu.VMEM((N,), f32),
                pltpu.VMEM((N,), f32),
                pltpu.VMEM((N,), f32),
                pltpu.VMEM((EPW,), f32),
                pltpu.VMEM((EPW,), f32),
                pltpu.VMEM((EPW,), f32),
            ]
            + [pltpu.SemaphoreType.DMA] * (2 * NBUF)
        ),
    )(_sc_gather_body)


def kernel(coord, feat, W1, W3, kernel_points, kp_weights,
           g1, b1, g2, b2, g3, b3, reference_index):
    f32 = jnp.float32
    g1 = g1.reshape(1, C); b1 = b1.reshape(1, C)
    g2 = g2.reshape(1, C); b2 = b2.reshape(1, C)
    g3 = g3.reshape(1, C); b3 = b3.reshape(1, C)

    x = pl.pallas_call(
        _fc1_bn_relu_body,
        out_shape=jax.ShapeDtypeStruct((N, C), f32),
    )(feat, W1.T, g1, b1)

    idx = reference_index.astype(jnp.int32).reshape(NW, NCHUNK, CHUNK)
    ct = coord.T
    cx, cy, cz = ct[0], ct[1], ct[2]
    gx, rel1d = _make_sc_gather()(x, cx, cy, cz, idx)

    kn2 = jnp.sum(kernel_points * kernel_points, axis=1)
    m5 = jnp.zeros((8, KP), f32)
    m5 = m5.at[0:3, :K].set(-2.0 * kernel_points.T)
    m5 = m5.at[3, :].set(1.0)
    m5 = m5.at[4, :K].set(kn2)
    kpw = jnp.zeros((KP, C), f32).at[:K, :].set(kp_weights)
    conv = pl.pallas_call(
        _kpconv_body,
        grid=(GB,),
        in_specs=[
            pl.BlockSpec((NB * H, C), lambda i: (i, 0)),
            pl.BlockSpec((3, NB * H), lambda i: (0, i)),
            pl.BlockSpec((8, KP), lambda i: (0, 0)),
            pl.BlockSpec((KP, C), lambda i: (0, 0)),
        ],
        out_specs=pl.BlockSpec((NB, C), lambda i: (i, 0)),
        out_shape=jax.ShapeDtypeStruct((N, C), f32),
    )(gx, rel1d.reshape(3, E), m5, kpw)

    out = pl.pallas_call(
        _epilogue_body,
        out_shape=jax.ShapeDtypeStruct((N, C), f32),
    )(conv, feat, W3.T, g2, b2, g3, b3)
    return out


# speedup vs baseline: 2.2419x; 1.4124x over previous
import functools

import jax
import jax.numpy as jnp
from jax import lax
from jax.experimental import pallas as pl
from jax.experimental.pallas import tpu as pltpu
from jax.experimental.pallas import tpu_sc as plsc

N = 10000
H = 32
C = 128
K = 43
KP = 48
E = N * H

NW = 32
CHUNK = 80
NCHUNK = E // (NW * CHUNK)
EPW = E // NW
NBUF = 5

NB = 400
GB = N // NB


def _fc1_bn_relu_body(feat_ref, w_ref, g_ref, b_ref, o_ref):
    y = jnp.dot(feat_ref[...], w_ref[...], preferred_element_type=jnp.float32)
    m = jnp.mean(y, axis=0, keepdims=True)
    v = jnp.mean((y - m) * (y - m), axis=0, keepdims=True)
    xn = (y - m) * lax.rsqrt(v + 1e-5) * g_ref[...] + b_ref[...]
    o_ref[...] = jnp.maximum(xn, 0.0)


def _epilogue_body(conv_ref, feat_ref, w_ref, g2_ref, b2_ref, g3_ref, b3_ref,
                   o_ref):
    y = conv_ref[...]
    m = jnp.mean(y, axis=0, keepdims=True)
    v = jnp.mean((y - m) * (y - m), axis=0, keepdims=True)
    y = jnp.maximum((y - m) * lax.rsqrt(v + 1e-5) * g2_ref[...] + b2_ref[...],
                    0.0)
    z = jnp.dot(y, w_ref[...], preferred_element_type=jnp.float32)
    m = jnp.mean(z, axis=0, keepdims=True)
    v = jnp.mean((z - m) * (z - m), axis=0, keepdims=True)
    z = (z - m) * lax.rsqrt(v + 1e-5) * g3_ref[...] + b3_ref[...]
    o_ref[...] = jnp.maximum(feat_ref[...] + z, 0.0)


def _kpconv_body(gx_ref, rel_ref, m5_ref, kpw_ref, o_ref):
    rel = rel_ref[...]
    sq = jnp.sum(rel * rel, axis=0, keepdims=True)
    r5 = jnp.concatenate(
        [rel, sq, jnp.ones((1, NB * H), jnp.float32)], axis=0)
    d2 = jax.lax.dot_general(
        r5, m5_ref[0:5, :], (((0,), (0,)), ((), ())),
        preferred_element_type=jnp.float32)
    d2 = jnp.maximum(d2, 0.0) + 1e-12
    infl = jnp.maximum(1.0 - d2 * jax.lax.rsqrt(d2), 0.0)
    a = jnp.dot(infl, kpw_ref[...], preferred_element_type=jnp.float32)
    contrib = a * gx_ref[...]
    o_ref[...] = jnp.sum(contrib.reshape(NB, H, C), axis=1)


def _sc_gather_body(x_hbm, cx_hbm, cy_hbm, cz_hbm, idx_hbm,
                    gx_hbm, rel_hbm,
                    idx_v, xrow_v, cx_v, cy_v, cz_v, rx_v, ry_v, rz_v, *sems):
    gsems = sems[:NBUF]
    wsems = sems[NBUF:]
    c = lax.axis_index("c")
    s = lax.axis_index("s")
    wid = s * 2 + c
    pltpu.sync_copy(idx_hbm.at[wid], idx_v)
    pltpu.sync_copy(cx_hbm, cx_v)
    pltpu.sync_copy(cy_hbm, cy_v)
    pltpu.sync_copy(cz_hbm, cz_v)
    lanes = lax.iota(jnp.int32, 16)
    wbase = pl.multiple_of(wid * EPW, 8)

    def g_start(b, i):
        pltpu.async_copy(x_hbm.at[idx_v.at[i]], xrow_v.at[b], gsems[b])

    for b in range(NBUF):
        g_start(b, b)

    def body(i5, carry):
        for b in range(NBUF):
            i = i5 * NBUF + b
            for v in range(CHUNK // 16):
                j = idx_v[i, pl.ds(v * 16, 16)]
                n = jax.lax.shift_right_logical(
                    wbase + i * CHUNK + v * 16 + lanes, 5)
                off = i * CHUNK + v * 16
                rx_v[pl.ds(off, 16)] = (plsc.load_gather(cx_v, [j])
                                        - plsc.load_gather(cx_v, [n]))
                ry_v[pl.ds(off, 16)] = (plsc.load_gather(cy_v, [j])
                                        - plsc.load_gather(cy_v, [n]))
                rz_v[pl.ds(off, 16)] = (plsc.load_gather(cz_v, [j])
                                        - plsc.load_gather(cz_v, [n]))
            pltpu.make_async_copy(
                x_hbm.at[pl.ds(0, CHUNK)], xrow_v.at[b], gsems[b]).wait()
            pltpu.async_copy(
                xrow_v.at[b],
                gx_hbm.at[pl.ds(wbase + i * CHUNK, CHUNK)], wsems[b])
        for b in range(NBUF):
            i_next = (i5 + 1) * NBUF + b
            pltpu.make_async_copy(
                xrow_v.at[b], gx_hbm.at[pl.ds(0, CHUNK)], wsems[b]).wait()

            @pl.when(i_next < NCHUNK)
            def _():
                g_start(b, i_next)

        return carry

    lax.fori_loop(0, NCHUNK // NBUF, body, 0)
    pltpu.sync_copy(rx_v, rel_hbm.at[pl.ds(wbase, EPW)])
    pltpu.sync_copy(ry_v, rel_hbm.at[pl.ds(E + wbase, EPW)])
    pltpu.sync_copy(rz_v, rel_hbm.at[pl.ds(2 * E + wbase, EPW)])


def _make_sc_gather():
    f32 = jnp.float32
    return functools.partial(
        pl.kernel,
        out_type=[
            jax.ShapeDtypeStruct((E, C), f32),
            jax.ShapeDtypeStruct((3 * E,), f32),
        ],
        mesh=plsc.VectorSubcoreMesh(core_axis_name="c", subcore_axis_name="s"),
        compiler_params=pltpu.CompilerParams(needs_layout_passes=False),
        scratch_types=(
            [
                pltpu.VMEM((NCHUNK, CHUNK), jnp.int32),
                pltpu.VMEM((NBUF, CHUNK, C), f32),
                pltpu.VMEM((N,), f32),
                pltpu.VMEM((N,), f32),
                pltpu.VMEM((N,), f32),
                pltpu.VMEM((EPW,), f32),
                pltpu.VMEM((EPW,), f32),
                pltpu.VMEM((EPW,), f32),
            ]
            + [pltpu.SemaphoreType.DMA] * (2 * NBUF)
        ),
    )(_sc_gather_body)


def kernel(coord, feat, W1, W3, kernel_points, kp_weights,
           g1, b1, g2, b2, g3, b3, reference_index):
    f32 = jnp.float32
    g1 = g1.reshape(1, C); b1 = b1.reshape(1, C)
    g2 = g2.reshape(1, C); b2 = b2.reshape(1, C)
    g3 = g3.reshape(1, C); b3 = b3.reshape(1, C)

    x = pl.pallas_call(
        _fc1_bn_relu_body,
        out_shape=jax.ShapeDtypeStruct((N, C), f32),
    )(feat, W1.T, g1, b1)

    idx = reference_index.astype(jnp.int32).reshape(NW, NCHUNK, CHUNK)
    ct = coord.T
    cx, cy, cz = ct[0], ct[1], ct[2]
    gx, rel1d = _make_sc_gather()(x, cx, cy, cz, idx)

    kn2 = jnp.sum(kernel_points * kernel_points, axis=1)
    m5 = jnp.zeros((8, KP), f32)
    m5 = m5.at[0:3, :K].set(-2.0 * kernel_points.T)
    m5 = m5.at[3, :].set(1.0)
    m5 = m5.at[4, :K].set(kn2)
    kpw = jnp.zeros((KP, C), f32).at[:K, :].set(kp_weights)
    conv = pl.pallas_call(
        _kpconv_body,
        grid=(GB,),
        in_specs=[
            pl.BlockSpec((NB * H, C), lambda i: (i, 0)),
            pl.BlockSpec((3, NB * H), lambda i: (0, i)),
            pl.BlockSpec((8, KP), lambda i: (0, 0)),
            pl.BlockSpec((KP, C), lambda i: (0, 0)),
        ],
        out_specs=pl.BlockSpec((NB, C), lambda i: (i, 0)),
        out_shape=jax.ShapeDtypeStruct((N, C), f32),
    )(gx, rel1d.reshape(3, E), m5, kpw)

    out = pl.pallas_call(
        _epilogue_body,
        out_shape=jax.ShapeDtypeStruct((N, C), f32),
    )(conv, feat, W3.T, g2, b2, g3, b3)
    return out
